# baseline (speedup 1.0000x reference)
import functools

import jax
import jax.numpy as jnp
from jax import lax
from jax.experimental import pallas as pl
from jax.experimental.pallas import tpu as pltpu
from jax.experimental.pallas import tpu_sc as plsc

NC = 2
NS = 16
NW = NC * NS

BT = 128
DT = 8


@functools.partial(jax.jit, static_argnames=("b", "s", "d"))
def _gather_t(x_r, table, *, b, s, d):
    nbt = b // BT
    bt_per_w = nbt // NW
    ndt = d // DT

    @functools.partial(
        pl.kernel,
        out_type=jax.ShapeDtypeStruct((s, ndt, nbt, DT, BT), jnp.float32),
        mesh=plsc.VectorSubcoreMesh(core_axis_name="c", subcore_axis_name="s"),
        scratch_types=[
            pltpu.VMEM((2, bt_per_w, BT), jnp.int32),
            pltpu.VMEM((2, bt_per_w, BT, d), jnp.float32),
            pltpu.VMEM((bt_per_w, ndt, DT, BT), jnp.float32),
            pltpu.SemaphoreType.DMA,
            pltpu.SemaphoreType.DMA,
        ],
        compiler_params=pltpu.CompilerParams(
            use_tc_tiling_on_sc=False, needs_layout_passes=False
        ),
    )
    def body(x_hbm, table_hbm, out_hbm, idx_v, rows_v, blk_v, gsem, osem):
        wid = lax.axis_index("s") * NC + lax.axis_index("c")
        lane = lax.iota(jnp.int32, 16)

        def load_idx(si, p):
            pltpu.sync_copy(x_hbm.at[si, wid], idx_v.at[p])

        def fire_gathers(p):
            return [
                pltpu.async_copy(
                    table_hbm.at[idx_v.at[p, bb]],
                    rows_v.at[p, bb],
                    gsem,
                )
                for bb in range(bt_per_w)
            ]

        def drain_gathers(p):
            for bb in range(bt_per_w):
                pltpu.make_async_copy(
                    table_hbm.at[idx_v.at[p, bb]], rows_v.at[p, bb], gsem
                ).wait()

        def fire_store(si, bb):
            pltpu.async_copy(
                blk_v.at[bb], out_hbm.at[si, :, wid * bt_per_w + bb], osem
            )

        def wait_store(bb):
            pltpu.make_async_copy(blk_v.at[bb], out_hbm.at[0, :, 0], osem).wait()

        def transpose_tile(p, bb):
            bis = [g * 16 + lane for g in range(BT // 16)]
            rows_s = rows_v.at[p, bb]
            blk_s = blk_v.at[bb]

            def d_body(dt2, _):
                d0 = dt2 * 16
                for k in range(16):
                    rot = jnp.bitwise_and(lane + k, 15)
                    dvec = d0 + rot
                    dt_i = jnp.right_shift(dvec, 3)
                    di_i = jnp.bitwise_and(dvec, 7)
                    vs = [
                        plsc.load_gather(rows_s, [bi, dvec])
                        for bi in bis
                    ]
                    for bi, v in zip(bis, vs):
                        plsc.store_scatter(blk_s, [dt_i, di_i, bi], v)
                return _

            lax.fori_loop(0, d // 16, d_body, 0)

        load_idx(0, jnp.int32(0))
        fire_gathers(jnp.int32(0))

        def step(si, carry):
            p = jnp.bitwise_and(si, 1)
            q = 1 - p

            @pl.when(si < s - 1)
            def _():
                load_idx(si + 1, q)
                fire_gathers(q)

            drain_gathers(p)

            for bb in range(bt_per_w):
                @pl.when(si > 0)
                def _(bb=bb):
                    wait_store(bb)

                transpose_tile(p, bb)
                fire_store(si, bb)
            return carry

        lax.fori_loop(0, s, step, 0)
        for bb in range(bt_per_w):
            wait_store(bb)

    return body(x_r, table)


def kernel(x, table):
    b, s = x.shape
    v, d = table.shape
    x_r = jnp.transpose(x.astype(jnp.int32)).reshape(s, NW, b // BT // NW, BT)
    t = _gather_t(x_r, table, b=b, s=s, d=d)
    return jnp.reshape(
        jnp.transpose(t, (2, 4, 0, 1, 3)), (b, s, d)
    )


# speedup vs baseline: 1.1458x; 1.1458x over previous
import functools

import jax
import jax.numpy as jnp
from jax import lax
from jax.experimental import pallas as pl
from jax.experimental.pallas import tpu as pltpu
from jax.experimental.pallas import tpu_sc as plsc

NC = 2
NS = 16
NW = NC * NS

BT = 128
DT = 8


@functools.partial(jax.jit, static_argnames=("b", "s", "d"))
def _gather_t(x_r, table, *, b, s, d):
    nbt = b // BT
    bt_per_w = nbt // NW
    ndt = d // DT

    @functools.partial(
        pl.kernel,
        out_type=jax.ShapeDtypeStruct((s, ndt, nbt, DT, BT), jnp.float32),
        mesh=plsc.VectorSubcoreMesh(core_axis_name="c", subcore_axis_name="s"),
        scratch_types=[
            pltpu.VMEM((2, bt_per_w, BT), jnp.int32),
            pltpu.VMEM((2, bt_per_w, BT, d), jnp.float32),
            pltpu.VMEM((bt_per_w, ndt, DT, BT), jnp.float32),
            pltpu.SemaphoreType.DMA,
            pltpu.SemaphoreType.DMA,
        ],
        compiler_params=pltpu.CompilerParams(
            use_tc_tiling_on_sc=False, needs_layout_passes=False
        ),
    )
    def body(x_hbm, table_hbm, out_hbm, idx_v, rows_v, blk_v, gsem, osem):
        wid = lax.axis_index("s") * NC + lax.axis_index("c")
        lane = lax.iota(jnp.int32, 16)

        def load_idx(si, p):
            pltpu.sync_copy(x_hbm.at[si, wid], idx_v.at[p])

        def fire_gathers(p):
            return [
                pltpu.async_copy(
                    table_hbm.at[idx_v.at[p, bb]],
                    rows_v.at[p, bb],
                    gsem,
                )
                for bb in range(bt_per_w)
            ]

        def drain_gathers(p):
            for bb in range(bt_per_w):
                pltpu.make_async_copy(
                    table_hbm.at[idx_v.at[p, bb]], rows_v.at[p, bb], gsem
                ).wait()

        def fire_store(si, bb):
            pltpu.async_copy(
                blk_v.at[bb], out_hbm.at[si, :, wid * bt_per_w + bb], osem
            )

        def wait_store(bb):
            pltpu.make_async_copy(blk_v.at[bb], out_hbm.at[0, :, 0], osem).wait()

        def transpose_tile(p, bb):
            bis = [g * 16 + lane for g in range(BT // 16)]
            rows_s = rows_v.at[p, bb]
            blk_s = blk_v.at[bb]

            def d_body(dt2, _):
                d0 = dt2 * 16
                for k in range(16):
                    rot = jnp.bitwise_and(lane + k, 15)
                    dvec = d0 + rot
                    dt_i = jnp.right_shift(dvec, 3)
                    di_i = jnp.bitwise_and(dvec, 7)
                    vs = [
                        plsc.load_gather(rows_s, [bi, dvec])
                        for bi in bis
                    ]
                    for bi, v in zip(bis, vs):
                        plsc.store_scatter(blk_s, [dt_i, di_i, bi], v)
                return _

            lax.fori_loop(0, d // 16, d_body, 0)

        load_idx(0, jnp.int32(0))
        fire_gathers(jnp.int32(0))

        def step(si, carry):
            p = jnp.bitwise_and(si, 1)
            q = 1 - p

            @pl.when(si < s - 1)
            def _():
                load_idx(si + 1, q)

            drain_gathers(p)

            @pl.when(si < s - 1)
            def _():
                fire_gathers(q)

            for bb in range(bt_per_w):
                @pl.when(si > 0)
                def _(bb=bb):
                    wait_store(bb)

                transpose_tile(p, bb)
                fire_store(si, bb)
            return carry

        lax.fori_loop(0, s, step, 0)
        for bb in range(bt_per_w):
            wait_store(bb)

    return body(x_r, table)


def kernel(x, table):
    b, s = x.shape
    v, d = table.shape
    x_r = jnp.transpose(x.astype(jnp.int32)).reshape(s, NW, b // BT // NW, BT)
    t = _gather_t(x_r, table, b=b, s=s, d=d)
    return jnp.reshape(
        jnp.transpose(t, (2, 4, 0, 1, 3)), (b, s, d)
    )
